# Initial kernel scaffold; baseline (speedup 1.0000x reference)
import jax
import jax.numpy as jnp
from jax.experimental import pallas as pl


def kernel(X, tables, kernel):
    raise NotImplementedError("write your pallas kernel here")


# speedup vs baseline: 4.2854x; 4.2854x over previous
import functools

import jax
import jax.numpy as jnp
from jax import lax
from jax.experimental import pallas as pl
from jax.experimental.pallas import tpu as pltpu
from jax.experimental.pallas import tpu_sc as plsc

B = 4096
F = 26
V = 100000
D = 64
PAIRS = F * (F - 1) // 2
FD = F * D
OUTW = FD + 2 * PAIRS
BF = B * F

NC = 2
NS = 16
NW = NC * NS
ROWS_PER_W = BF // NW
GCH = 128
NG = ROWS_PER_W // GCH

BB = 256


def _sc_gather(idx3, tbl):
    @functools.partial(
        pl.kernel,
        mesh=plsc.VectorSubcoreMesh(core_axis_name="c", subcore_axis_name="s"),
        out_type=jax.ShapeDtypeStruct((BF, D), jnp.float32),
        scratch_types=[
            pltpu.VMEM((NG, GCH), jnp.int32),
            pltpu.VMEM((GCH, D), jnp.float32),
            pltpu.SemaphoreType.DMA,
        ],
        compiler_params=pltpu.CompilerParams(use_tc_tiling_on_sc=False),
    )
    def k(idx_hbm, tbl_hbm, out_hbm, idx_v, buf, sem):
        wid = lax.axis_index("s") * NC + lax.axis_index("c")
        pltpu.sync_copy(idx_hbm.at[wid], idx_v)
        base = wid * ROWS_PER_W

        def body(j, carry):
            pltpu.async_copy(tbl_hbm.at[idx_v.at[j]], buf, sem).wait()
            pltpu.sync_copy(buf, out_hbm.at[pl.ds(base + j * GCH, GCH)])
            return carry

        lax.fori_loop(0, NG, body, 0)

    return k(idx3, tbl)


def _tc_body(e_ref, w_ref, s_ref, out_ref):
    E = e_ref[...]
    out_ref[:, 0:FD] = E
    inner_parts, outer_parts = [], []
    ps = 0
    dnums = (((1,), (0,)), ((), ()))
    for i in range(F - 1):
        n = F - 1 - i
        p = E[:, i * D:(i + 1) * D]
        q = E[:, (i + 1) * D:FD]
        w = w_ref[:, ps * D:(ps + n) * D]
        t = lax.dot_general(p, w, dnums, preferred_element_type=jnp.float32)
        s = s_ref[0:n * D, 0:n]
        pt = jnp.concatenate([p] * n, axis=1)
        inner_parts.append(
            lax.dot_general(pt * q, s, dnums, preferred_element_type=jnp.float32))
        outer_parts.append(
            lax.dot_general(t * q, s, dnums, preferred_element_type=jnp.float32))
        ps += n
    out_ref[:, FD:FD + PAIRS] = jnp.concatenate(inner_parts, axis=1)
    out_ref[:, FD + PAIRS:OUTW] = jnp.concatenate(outer_parts, axis=1)


def _tc_interact(E, Wbig, S):
    return pl.pallas_call(
        _tc_body,
        grid=(B // BB,),
        in_specs=[
            pl.BlockSpec((BB, FD), lambda b: (b, 0)),
            pl.BlockSpec((D, PAIRS * D), lambda b: (0, 0)),
            pl.BlockSpec(((F - 1) * D, F - 1), lambda b: (0, 0)),
        ],
        out_specs=pl.BlockSpec((BB, OUTW), lambda b: (b, 0)),
        out_shape=jax.ShapeDtypeStruct((B, OUTW), jnp.float32),
    )(E, Wbig, S)


def kernel(X, tables, kern):
    idx = (X.astype(jnp.int32)
           + (jnp.arange(F, dtype=jnp.int32) * V)[None, :]).reshape(NW, NG, GCH)
    tbl = tables.reshape(F * V, D)
    emb = _sc_gather(idx, tbl)
    E = emb.reshape(B, FD)
    Wbig = kern.transpose(2, 1, 0).reshape(D, PAIRS * D)
    S = (jnp.arange((F - 1) * D, dtype=jnp.int32)[:, None] // D
         == jnp.arange(F - 1, dtype=jnp.int32)[None, :]).astype(jnp.float32)
    return _tc_interact(E, Wbig, S)
